# baseline (speedup 1.0000x reference)
import functools

import jax
import jax.numpy as jnp
from jax import lax
from jax.experimental import pallas as pl
from jax.experimental.pallas import tpu as pltpu
from jax.experimental.pallas import tpu_sc as plsc

_MIN_DISABLED = 32
_MAX_DISABLED = 128
_CHUNK_ROWS = 16
_VEC = 16


@functools.cache
def _disabled_span(tof_count: int) -> tuple[int, int]:
    with jax.ensure_compile_time_eval():
        key = jax.random.key(42)
        k1, k2 = jax.random.split(key)
        count = int(jax.random.randint(k1, (), _MIN_DISABLED, _MAX_DISABLED + 1))
        start = int(jax.random.randint(k2, (), 0, tof_count))
    return start, count


def _mask_plan(start, count, tof_count):
    end = start + count
    disabled = [((c - start) % tof_count) < count for c in range(tof_count)]
    plan = []
    for v0 in range(0, tof_count, _VEC):
        cols = disabled[v0:v0 + _VEC]
        if all(cols):
            plan.append((v0, "zero", 0, _VEC))
        elif any(cols):
            lo = cols.index(True)
            hi = _VEC - cols[::-1].index(True)
            assert all(cols[lo:hi]) and not any(cols[:lo]) and not any(cols[hi:])
            plan.append((v0, "rmw", lo, hi))
    return plan


def kernel(img):
    rows, tof_count = img.shape
    start, count = _disabled_span(tof_count)
    plan = _mask_plan(start, count, tof_count)

    nc, ns = 2, 16
    nw = nc * ns
    rows_per = rows // nw
    n_chunks = rows_per // _CHUNK_ROWS
    mesh = plsc.VectorSubcoreMesh(core_axis_name="c", subcore_axis_name="s")

    @functools.partial(
        pl.kernel,
        mesh=mesh,
        out_type=jax.ShapeDtypeStruct((rows, tof_count), jnp.float32),
        scratch_types=[
            pltpu.VMEM((_CHUNK_ROWS, tof_count), jnp.float32),
            pltpu.VMEM((_CHUNK_ROWS, tof_count), jnp.float32),
            pltpu.SemaphoreType.DMA,
            pltpu.SemaphoreType.DMA,
            pltpu.SemaphoreType.DMA,
            pltpu.SemaphoreType.DMA,
        ],
    )
    def k(img_hbm, out_hbm, buf0, buf1, rs0, rs1, ws0, ws1):
        wid = lax.axis_index("s") * nc + lax.axis_index("c")
        base = wid * rows_per
        bufs = (buf0, buf1)
        rsems = (rs0, rs1)
        wsems = (ws0, ws1)

        def rd(c):
            return pltpu.async_copy(
                img_hbm.at[pl.ds(base + c * _CHUNK_ROWS, _CHUNK_ROWS)],
                bufs[c % 2], rsems[c % 2])

        def wr(c):
            return pltpu.async_copy(
                bufs[c % 2],
                out_hbm.at[pl.ds(base + c * _CHUNK_ROWS, _CHUNK_ROWS)],
                wsems[c % 2])

        def apply_mask(buf):
            zeros = jnp.zeros((_VEC,), jnp.float32)
            lane = lax.broadcasted_iota(jnp.int32, (_VEC,), 0)
            for r in range(_CHUNK_ROWS):
                for v0, kind, lo, hi in plan:
                    if kind == "zero":
                        buf[r, pl.ds(v0, _VEC)] = zeros
                    else:
                        keep = (lane < lo) | (lane >= hi)
                        cur = buf[r, pl.ds(v0, _VEC)]
                        buf[r, pl.ds(v0, _VEC)] = jnp.where(keep, cur, zeros)

        reads = {}
        writes = {}
        reads[0] = rd(0)
        for c in range(n_chunks):
            nxt = c + 1
            if nxt < n_chunks:
                if nxt >= 2:
                    writes[nxt - 2].wait()
                reads[nxt] = rd(nxt)
            reads[c].wait()
            apply_mask(bufs[c % 2])
            writes[c] = wr(c)
        writes[n_chunks - 2].wait()
        writes[n_chunks - 1].wait()

    return k(img)


# speedup vs baseline: 1.4106x; 1.4106x over previous
import functools

import jax
import jax.numpy as jnp
from jax.experimental import pallas as pl
from jax.experimental.pallas import tpu as pltpu

_MIN_DISABLED = 32
_MAX_DISABLED = 128
_LANE = 128
_N_CHUNKS = 16
_N_BUF = 4


@functools.cache
def _disabled_span(tof_count: int) -> tuple[int, int]:
    with jax.ensure_compile_time_eval():
        key = jax.random.key(42)
        k1, k2 = jax.random.split(key)
        count = int(jax.random.randint(k1, (), _MIN_DISABLED, _MAX_DISABLED + 1))
        start = int(jax.random.randint(k2, (), 0, tof_count))
    return start, count


def _bounce_body(img_ref, out_ref, *rest, masked_tiles, start, count,
                 tof_count):
    bufs = rest[:_N_BUF]
    rsems = rest[_N_BUF:_N_BUF + _N_CHUNKS]
    wsems = rest[_N_BUF + _N_CHUNKS:]
    rows = img_ref.shape[0]
    chunk_rows = rows // _N_CHUNKS

    def read_cp(c):
        return pltpu.make_async_copy(
            img_ref.at[c * chunk_rows:(c + 1) * chunk_rows],
            bufs[c % _N_BUF], rsems[c])

    def write_cp(c):
        return pltpu.make_async_copy(
            bufs[c % _N_BUF],
            out_ref.at[c * chunk_rows:(c + 1) * chunk_rows], wsems[c])

    reads = {}
    writes = {}
    reads[0] = read_cp(0)
    reads[0].start()
    for c in range(_N_CHUNKS):
        nxt = c + 1
        if nxt < _N_CHUNKS:
            if nxt >= _N_BUF:
                writes[nxt - _N_BUF].wait()
            reads[nxt] = read_cp(nxt)
            reads[nxt].start()
        reads[c].wait()
        buf = bufs[c % _N_BUF]
        for t in masked_tiles:
            strip = buf[:, t * _LANE:(t + 1) * _LANE]
            cols = t * _LANE + jax.lax.broadcasted_iota(
                jnp.int32, strip.shape, 1)
            disabled = ((cols - start) % tof_count) < count
            buf[:, t * _LANE:(t + 1) * _LANE] = jnp.where(
                disabled, jnp.float32(0.0), strip)
        writes[c] = write_cp(c)
        writes[c].start()
    for c in range(max(0, _N_CHUNKS - _N_BUF), _N_CHUNKS):
        writes[c].wait()


def kernel(img):
    rows, tof_count = img.shape
    start, count = _disabled_span(tof_count)
    end = start + count

    n_tiles = tof_count // _LANE
    t0 = start // _LANE
    t1 = ((end - 1) // _LANE) % n_tiles
    masked_tiles = sorted({t0, t1})

    chunk_rows = rows // _N_CHUNKS
    body = functools.partial(
        _bounce_body, masked_tiles=masked_tiles,
        start=start, count=count, tof_count=tof_count)
    return pl.pallas_call(
        body,
        in_specs=[pl.BlockSpec(memory_space=pl.ANY)],
        out_specs=pl.BlockSpec(memory_space=pl.ANY),
        out_shape=jax.ShapeDtypeStruct((rows, tof_count), jnp.float32),
        scratch_shapes=(
            [pltpu.VMEM((chunk_rows, tof_count), jnp.float32)
             for _ in range(_N_BUF)]
            + [pltpu.SemaphoreType.DMA for _ in range(2 * _N_CHUNKS)]
        ),
    )(img)
